# baseline (speedup 1.0000x reference)
import numpy as np
import jax
import jax.numpy as jnp
from jax.experimental import pallas as pl
from jax.experimental.pallas import tpu as pltpu

_NUM_CLASSES = 80
_NUM_ANCHORS = 9
_STRIDE = 8
_PRE_NMS = 1000
_MAX_OUT = 100
_IOU_TH = 0.5
_PAD = 1024


def _make_anchor_table(h, w, stride):
    base = stride * 4.0
    scales = np.array([2.0 ** 0, 2.0 ** (1.0 / 3.0), 2.0 ** (2.0 / 3.0)])
    ratios = np.array([0.5, 1.0, 2.0])
    ws, hs = [], []
    for r in ratios:
        for s in scales:
            ws.append(base * s / np.sqrt(r))
            hs.append(base * s * np.sqrt(r))
    ws = np.array(ws)
    hs = np.array(hs)
    sx = (np.arange(w) + 0.5) * stride
    sy = (np.arange(h) + 0.5) * stride
    cx, cy = np.meshgrid(sx, sy)
    cx = cx.reshape(-1, 1)
    cy = cy.reshape(-1, 1)
    a = np.stack([cx - ws / 2.0, cy - hs / 2.0, cx + ws / 2.0, cy + hs / 2.0],
                 axis=-1)
    return jnp.asarray(a.reshape(-1, 4), jnp.float32)


def _nms_kernel(boxes_ref, boxest_ref, clsc_ref, scr_ref, out_ref, smat_ref):
    n = _PAD
    nblk = n // 128

    b = boxes_ref[0]
    clsc = clsc_ref[0]
    bt = boxest_ref[0]
    s_row = scr_ref[0]

    ob = b + clsc * 1e4
    x1 = ob[:, 0:1]
    y1 = ob[:, 1:2]
    x2 = ob[:, 2:3]
    y2 = ob[:, 3:4]
    area = jnp.maximum(x2 - x1, 0.0) * jnp.maximum(y2 - y1, 0.0)

    clsr = bt[4:5, :]
    x1r = bt[0:1, :] + clsr * 1e4
    y1r = bt[1:2, :] + clsr * 1e4
    x2r = bt[2:3, :] + clsr * 1e4
    y2r = bt[3:4, :] + clsr * 1e4
    arear = jnp.maximum(x2r - x1r, 0.0) * jnp.maximum(y2r - y1r, 0.0)

    jlane = jax.lax.broadcasted_iota(jnp.int32, (128, n), 1)
    for rb in range(nblk):
        sl = slice(rb * 128, rb * 128 + 128)
        xx1 = jnp.maximum(x1[sl, :], x1r)
        yy1 = jnp.maximum(y1[sl, :], y1r)
        xx2 = jnp.minimum(x2[sl, :], x2r)
        yy2 = jnp.minimum(y2[sl, :], y2r)
        inter = jnp.maximum(xx2 - xx1, 0.0) * jnp.maximum(yy2 - yy1, 0.0)
        iou = inter / (area[sl, :] + arear - inter + 1e-9)
        irow = jax.lax.broadcasted_iota(jnp.int32, (128, n), 0) + rb * 128
        mask = jnp.logical_and(iou > _IOU_TH, jlane > irow)
        smat_ref[pl.ds(rb * 128, 128), :] = mask.astype(jnp.float32)

    lane = jax.lax.broadcasted_iota(jnp.int32, (1, n), 1)

    def greedy(i, valid):
        row = smat_ref[pl.ds(i, 1), :]
        vi = jnp.sum(jnp.where(lane == i, valid, 0.0), axis=1, keepdims=True)
        return valid * (1.0 - row * vi)

    valid = jax.lax.fori_loop(0, n, greedy, jnp.ones((1, n), jnp.float32))

    s_cur = jnp.where(valid > 0.5, s_row, -1e9)
    riota = jax.lax.broadcasted_iota(jnp.int32, (128, 1), 0)
    zero_col = jnp.zeros((128, 1), jnp.float32)

    def pick(k, carry):
        s_c, ox1, oy1, ox2, oy2, osc, ocl = carry
        m = jnp.max(s_c, axis=1, keepdims=True)
        sel = s_c == m
        idx = jnp.min(jnp.where(sel, lane, n), axis=1, keepdims=True)
        oneh = (lane == idx).astype(jnp.float32)
        kcol = (riota == k).astype(jnp.float32)

        def take(rowvec):
            return jnp.sum(oneh * rowvec, axis=1, keepdims=True)

        ox1 = ox1 + kcol * take(bt[0:1, :])
        oy1 = oy1 + kcol * take(bt[1:2, :])
        ox2 = ox2 + kcol * take(bt[2:3, :])
        oy2 = oy2 + kcol * take(bt[3:4, :])
        osc = osc + kcol * m
        ocl = ocl + kcol * take(clsr)
        s_c = jnp.where(lane == idx, -3e38, s_c)
        return s_c, ox1, oy1, ox2, oy2, osc, ocl

    init = (s_cur, zero_col, zero_col, zero_col, zero_col, zero_col, zero_col)
    _, ox1, oy1, ox2, oy2, osc, ocl = jax.lax.fori_loop(0, _MAX_OUT, pick, init)

    out_ref[0] = jnp.concatenate(
        [ox1, oy1, ox2, oy2, osc, ocl, zero_col, zero_col], axis=1)


def kernel(cls_pred, loc_pred):
    B, _, H, W = cls_pred.shape
    K = H * W * _NUM_ANCHORS

    cls = jnp.transpose(cls_pred, (0, 2, 3, 1)).reshape(B, K, _NUM_CLASSES)
    loc = jnp.transpose(loc_pred, (0, 2, 3, 1)).reshape(B, K, 4)
    cls = jax.nn.sigmoid(cls)
    anchors = _make_anchor_table(H, W, _STRIDE)

    flat = cls.reshape(B, K * _NUM_CLASSES)
    top_s, top_i = jax.lax.top_k(flat, _PRE_NMS)
    anchor_idx = top_i // _NUM_CLASSES
    cls_idx = top_i % _NUM_CLASSES

    pw = anchors[:, 2] - anchors[:, 0]
    ph = anchors[:, 3] - anchors[:, 1]
    pcx = anchors[:, 0] + 0.5 * pw
    pcy = anchors[:, 1] + 0.5 * ph
    dx = loc[..., 0]
    dy = loc[..., 1]
    dw = jnp.clip(loc[..., 2], -4.135, 4.135)
    dh = jnp.clip(loc[..., 3], -4.135, 4.135)
    cx = dx * pw + pcx
    cy = dy * ph + pcy
    w = jnp.exp(dw) * pw
    h = jnp.exp(dh) * ph
    boxes_all = jnp.stack(
        [cx - 0.5 * w, cy - 0.5 * h, cx + 0.5 * w, cy + 0.5 * h], axis=-1)
    cand = jnp.take_along_axis(boxes_all, anchor_idx[..., None], axis=1)

    pad = _PAD - _PRE_NMS
    boxes_p = jnp.pad(cand, ((0, 0), (0, pad), (0, 0)))
    clsf = jnp.pad(cls_idx.astype(jnp.float32), ((0, 0), (0, pad)))
    scr = jnp.pad(top_s, ((0, 0), (0, pad)), constant_values=-1e9)

    boxest = jnp.concatenate(
        [jnp.transpose(boxes_p, (0, 2, 1)), clsf[:, None, :]], axis=1)

    out = pl.pallas_call(
        _nms_kernel,
        grid=(B,),
        in_specs=[
            pl.BlockSpec((1, _PAD, 4), lambda b: (b, 0, 0)),
            pl.BlockSpec((1, 5, _PAD), lambda b: (b, 0, 0)),
            pl.BlockSpec((1, _PAD, 1), lambda b: (b, 0, 0)),
            pl.BlockSpec((1, 1, _PAD), lambda b: (b, 0, 0)),
        ],
        out_specs=pl.BlockSpec((1, 128, 8), lambda b: (b, 0, 0)),
        out_shape=jax.ShapeDtypeStruct((B, 128, 8), jnp.float32),
        scratch_shapes=[pltpu.VMEM((_PAD, _PAD), jnp.float32)],
    )(boxes_p, boxest, clsf[..., None], scr[:, None, :])

    return out[:, :_MAX_OUT, :6]
